# Initial kernel scaffold; baseline (speedup 1.0000x reference)
import functools

import jax
import jax.numpy as jnp
from jax import lax
from jax.experimental import pallas as pl
from jax.experimental.pallas import tpu as pltpu
from jax.experimental.pallas import tpu_sc as plsc

H = 128
N = 10000
E = 320000
NG = 64
NLAYER = 3


def _inproj_body(x_ref, w_ref, b_ref, g_ref, bb_ref, o_ref):
    y = jnp.dot(x_ref[...], w_ref[...], preferred_element_type=jnp.float32)
    y = y + b_ref[...]
    m = jnp.mean(y, axis=0, keepdims=True)
    v = jnp.mean((y - m) ** 2, axis=0, keepdims=True)
    h = (y - m) * lax.rsqrt(v + 1e-5) * g_ref[...] + bb_ref[...]
    o_ref[...] = jnp.maximum(h, 0.0)


def _inproj(x, w, b, g, bb):
    return pl.pallas_call(
        _inproj_body,
        out_shape=jax.ShapeDtypeStruct((N, H), jnp.float32),
    )(x, w, b, g, bb)


_BE = 4000


def _edget_body(ea_ref, ew_ref, eb_ref, w0_ref, b0_ref, w1_ref, b1_ref,
                w2_ref, b2_ref, t0_ref, t1_ref, t2_ref):
    ea = jnp.dot(ea_ref[...], ew_ref[...], preferred_element_type=jnp.float32)
    ea = jnp.maximum(ea + eb_ref[...], 0.0)
    t0_ref[...] = jnp.dot(ea, w0_ref[...], preferred_element_type=jnp.float32) + b0_ref[...]
    t1_ref[...] = jnp.dot(ea, w1_ref[...], preferred_element_type=jnp.float32) + b1_ref[...]
    t2_ref[...] = jnp.dot(ea, w2_ref[...], preferred_element_type=jnp.float32) + b2_ref[...]


def _edge_t(edge_attr, ew, eb, ws, bs):
    grid = (E // _BE,)
    de = edge_attr.shape[1]
    wspec = pl.BlockSpec((de, H), lambda i: (0, 0))
    mspec = pl.BlockSpec((H, H), lambda i: (0, 0))
    bspec = pl.BlockSpec((1, H), lambda i: (0, 0))
    ospec = pl.BlockSpec((_BE, H), lambda i: (i, 0))
    return pl.pallas_call(
        _edget_body,
        grid=grid,
        in_specs=[pl.BlockSpec((_BE, de), lambda i: (i, 0)), wspec, bspec,
                  mspec, bspec, mspec, bspec, mspec, bspec],
        out_specs=[ospec, ospec, ospec],
        out_shape=[jax.ShapeDtypeStruct((E, H), jnp.float32)] * 3,
    )(edge_attr, ew, eb, ws[0], bs[0], ws[1], bs[1], ws[2], bs[2])


_NW = 32
_EPW = E // _NW
_KC = 80
_NCHUNK = _EPW // _KC
_RPT = N // 16
_RB = 125


def _edge_sc_body(h_hbm, t_hbm, src_hbm, dst_hbm, out_hbm,
                  src_v, dst_v, t_v, rows_v, zb_v, sem, aggr_sh):
    c = lax.axis_index("c")
    s = lax.axis_index("s")

    def _zrow(i, carry):
        for c8 in range(8):
            zb_v[i, pl.ds(c8 * 16, 16)] = jnp.zeros((16,), jnp.float32)
        return carry

    lax.fori_loop(0, _RB, _zrow, 0)
    for j in range(_RPT // _RB):
        pltpu.sync_copy(zb_v, aggr_sh.at[pl.ds(s * _RPT + j * _RB, _RB)])
    plsc.subcore_barrier()

    base0 = (c * 16 + s) * _EPW

    def _chunk(i, carry):
        base = base0 + i * _KC
        pltpu.sync_copy(src_hbm.at[pl.ds(base, _KC)], src_v)
        pltpu.sync_copy(dst_hbm.at[pl.ds(base, _KC)], dst_v)
        pltpu.sync_copy(t_hbm.at[pl.ds(base, _KC)], t_v)
        pltpu.async_copy(h_hbm.at[src_v], rows_v, sem).wait()

        def _row(r, rc):
            for c8 in range(8):
                sl = pl.ds(c8 * 16, 16)
                rows_v[r, sl] = jnp.maximum(rows_v[r, sl] + t_v[r, sl], 0.0)
            return rc

        lax.fori_loop(0, _KC, _row, 0)
        pltpu.sync_copy(rows_v, aggr_sh.at[dst_v], add=True)
        return carry

    lax.fori_loop(0, _NCHUNK, _chunk, 0)
    plsc.subcore_barrier()

    for j in range(_RPT // _RB):
        r0 = s * _RPT + j * _RB
        pltpu.sync_copy(aggr_sh.at[pl.ds(r0, _RB)], zb_v)
        pltpu.sync_copy(zb_v, out_hbm.at[c, pl.ds(r0, _RB)])


def _edge_pass(h, t, src, dst):
    mesh = plsc.VectorSubcoreMesh(core_axis_name="c", subcore_axis_name="s")
    f = pl.kernel(
        _edge_sc_body,
        out_type=jax.ShapeDtypeStruct((2, N, H), jnp.float32),
        mesh=mesh,
        scratch_types=[
            pltpu.VMEM((_KC,), jnp.int32),
            pltpu.VMEM((_KC,), jnp.int32),
            pltpu.VMEM((_KC, H), jnp.float32),
            pltpu.VMEM((_KC, H), jnp.float32),
            pltpu.VMEM((_RB, H), jnp.float32),
            pltpu.SemaphoreType.DMA,
            pltpu.VMEM_SHARED((N, H), jnp.float32),
        ],
    )
    return f(h, t, src, dst)


def _node_body(h_ref, a0_ref, a1_ref, sc_ref, w1_ref, b1_ref, g_ref, bb_ref,
               w2_ref, b2_ref, lg_ref, lb_ref, o_ref):
    h = h_ref[...]
    z = sc_ref[...] * h + a0_ref[...] + a1_ref[...]
    z = jnp.dot(z, w1_ref[...], preferred_element_type=jnp.float32) + b1_ref[...]
    m = jnp.mean(z, axis=0, keepdims=True)
    v = jnp.mean((z - m) ** 2, axis=0, keepdims=True)
    z = (z - m) * lax.rsqrt(v + 1e-5) * g_ref[...] + bb_ref[...]
    z = jnp.maximum(z, 0.0)
    z = jnp.dot(z, w2_ref[...], preferred_element_type=jnp.float32) + b2_ref[...]
    lm = jnp.mean(z, axis=-1, keepdims=True)
    lv = jnp.mean((z - lm) ** 2, axis=-1, keepdims=True)
    z = (z - lm) * lax.rsqrt(lv + 1e-5) * lg_ref[...] + lb_ref[...]
    o_ref[...] = jnp.maximum(z + h, 0.0)


def _node_mlp(h, aggr, scale, w1, b1, g, bb, w2, b2, lg, lb):
    return pl.pallas_call(
        _node_body,
        out_shape=jax.ShapeDtypeStruct((N, H), jnp.float32),
    )(h, aggr[0], aggr[1], scale, w1, b1, g, bb, w2, b2, lg, lb)


def _readout_body(h1_ref, h2_ref, h3_ref, bf_ref, gfp_ref,
                  jw1_ref, jb1_ref, jw2t_ref, jb2_ref,
                  wa_ref, wb_ref, wc_ref, wgf_ref, ball_ref,
                  cg_ref, cb_ref, w2t_ref, cb2_ref,
                  o_ref, mx_ref):
    hs = (h1_ref[...], h2_ref[...], h3_ref[...])
    scores = []
    for h in hs:
        t = jnp.dot(h, jw1_ref[...], preferred_element_type=jnp.float32)
        t = jnp.maximum(t + jb1_ref[...], 0.0)
        sc = jnp.dot(t, jw2t_ref[...], preferred_element_type=jnp.float32)
        scores.append(sc + jb2_ref[...])
    msc = jnp.maximum(jnp.maximum(scores[0], scores[1]), scores[2])
    e0 = jnp.exp(scores[0] - msc)
    e1 = jnp.exp(scores[1] - msc)
    e2 = jnp.exp(scores[2] - msc)
    inv = 1.0 / (e0 + e1 + e2)
    xf = (e0 * hs[0] + e1 * hs[1] + e2 * hs[2]) * inv

    bf = bf_ref[...]
    gids = lax.broadcasted_iota(jnp.float32, (1, NG), 1)
    mask = jnp.where(bf == gids, 1.0, 0.0)
    dn = (((0,), (0,)), ((), ()))
    ssum = lax.dot_general(mask, xf, dn, preferred_element_type=jnp.float32)
    ones = jnp.ones_like(xf)
    cnt = lax.dot_general(mask, ones, dn, preferred_element_type=jnp.float32)

    def _mx(g, carry):
        gf = lax.convert_element_type(g, jnp.float32)
        w = jnp.where(bf == gf, xf, -1e30)
        mx_ref[pl.ds(g, 1), :] = jnp.max(w, axis=0, keepdims=True)
        return carry

    lax.fori_loop(0, NG, _mx, 0)
    mx = jnp.where(cnt > 0.0, mx_ref[...], 0.0)
    mean = ssum / jnp.maximum(cnt, 1.0)

    pre = jnp.dot(mean, wa_ref[...], preferred_element_type=jnp.float32)
    pre = pre + jnp.dot(mx, wb_ref[...], preferred_element_type=jnp.float32)
    pre = pre + jnp.dot(ssum, wc_ref[...], preferred_element_type=jnp.float32)
    pre = pre + jnp.dot(gfp_ref[...], wgf_ref[...], preferred_element_type=jnp.float32)
    pre = pre + ball_ref[...]
    m = jnp.mean(pre, axis=0, keepdims=True)
    v = jnp.mean((pre - m) ** 2, axis=0, keepdims=True)
    z = (pre - m) * lax.rsqrt(v + 1e-5) * cg_ref[...] + cb_ref[...]
    z = jnp.maximum(z, 0.0)
    o_ref[...] = jnp.dot(z, w2t_ref[...], preferred_element_type=jnp.float32) + cb2_ref[...]


def _readout(h1, h2, h3, bf, gfp, jw1, jb1, jw2t, jb2, wa, wb, wc, wgf, ball,
             cg, cb, w2t, cb2):
    return pl.pallas_call(
        _readout_body,
        out_shape=jax.ShapeDtypeStruct((NG, H), jnp.float32),
        scratch_shapes=[pltpu.VMEM((NG, H), jnp.float32)],
    )(h1, h2, h3, bf, gfp, jw1, jb1, jw2t, jb2, wa, wb, wc, wgf, ball,
      cg, cb, w2t, cb2)


def _row(v):
    return v.reshape(1, -1).astype(jnp.float32)


def kernel(x, edge_index, edge_attr, batch, graph_features, params):
    p = params
    src = edge_index[0]
    dst = edge_index[1]

    h = _inproj(x, p['in_w'], _row(p['in_b']), _row(p['in_bn_g']),
                _row(p['in_bn_b']))

    ws = [bp['elin_w'] for bp in p['blocks']]
    bs = [_row(bp['elin_b']) for bp in p['blocks']]
    ts = _edge_t(edge_attr, p['e_w'], _row(p['e_b']), ws, bs)

    outs = []
    for l, bp in enumerate(p['blocks']):
        aggr = _edge_pass(h, ts[l], src, dst)
        scale = jnp.full((1, 1), 1.0, jnp.float32) + bp['eps']
        h = _node_mlp(h, aggr, scale, bp['w1'], _row(bp['b1']),
                      _row(bp['bn_g']), _row(bp['bn_b']), bp['w2'],
                      _row(bp['b2']), _row(bp['ln_g']), _row(bp['ln_b']))
        outs.append(h)

    bf = batch.astype(jnp.float32).reshape(N, 1)
    gfp = jnp.pad(graph_features, ((0, 0), (0, 6))).astype(jnp.float32)
    wgf_full = jnp.pad(p['gf_w'], ((0, 6), (0, 0)))

    wa = p['cls_w1'][0:H]
    wb = p['cls_w1'][H:2 * H]
    wc = p['cls_w1'][2 * H:3 * H]
    wd = p['cls_w1'][3 * H:]
    wgf = wgf_full @ wd
    ball = _row(p['cls_b1'] + p['gf_b'] @ wd)

    jw2t = jnp.tile(p['jk_w2'], (1, H))
    w2t = jnp.tile(p['cls_w2'], (1, H))

    out = _readout(outs[0], outs[1], outs[2], bf, gfp,
                   p['jk_w1'], _row(p['jk_b1']), jw2t,
                   jnp.full((1, 1), 1.0, jnp.float32) * p['jk_b2'],
                   wa, wb, wc, wgf, ball,
                   _row(p['cls_bn_g']), _row(p['cls_bn_b']), w2t,
                   jnp.full((1, 1), 1.0, jnp.float32) * p['cls_b2'])
    return out[:, 0]


# speedup vs baseline: 2.3349x; 2.3349x over previous
import functools

import jax
import jax.numpy as jnp
from jax import lax
from jax.experimental import pallas as pl
from jax.experimental.pallas import tpu as pltpu
from jax.experimental.pallas import tpu_sc as plsc

H = 128
N = 10000
E = 320000
NG = 64
NLAYER = 3


def _mm_in_body(x_ref, w_ref, b_ref, o_ref):
    o_ref[...] = jnp.dot(x_ref[...], w_ref[...],
                         preferred_element_type=jnp.float32) + b_ref[...]


def _mm_in(x, w, b):
    return pl.pallas_call(
        _mm_in_body,
        out_shape=jax.ShapeDtypeStruct((N, H), jnp.float32),
    )(x, w, b)


def _norm_relu_body(y_ref, m_ref, s_ref, g_ref, b_ref, o_ref):
    o_ref[...] = jnp.maximum(
        (y_ref[...] - m_ref[...]) / s_ref[...] * g_ref[...] + b_ref[...], 0.0)


def _norm_relu(y, m, s, g, b):
    return pl.pallas_call(
        _norm_relu_body,
        out_shape=jax.ShapeDtypeStruct((N, H), jnp.float32),
    )(y, m, s, g, b)


def _zw1_body(h_ref, a0_ref, a1_ref, sc_ref, w_ref, b_ref, o_ref):
    z = sc_ref[...] * h_ref[...] + (a0_ref[...] + a1_ref[...])
    o_ref[...] = jnp.dot(z, w_ref[...],
                         preferred_element_type=jnp.float32) + b_ref[...]


def _zw1(h, aggr, scale, w, b):
    return pl.pallas_call(
        _zw1_body,
        out_shape=jax.ShapeDtypeStruct((N, H), jnp.float32),
    )(h, aggr[0], aggr[1], scale, w, b)


def _nw2_body(y_ref, m_ref, s_ref, g_ref, bb_ref, w_ref, b_ref, o_ref):
    z = jnp.maximum(
        (y_ref[...] - m_ref[...]) / s_ref[...] * g_ref[...] + bb_ref[...], 0.0)
    o_ref[...] = jnp.dot(z, w_ref[...],
                         preferred_element_type=jnp.float32) + b_ref[...]


def _nw2(y, m, s, g, bb, w, b):
    return pl.pallas_call(
        _nw2_body,
        out_shape=jax.ShapeDtypeStruct((N, H), jnp.float32),
    )(y, m, s, g, bb, w, b)


def _lnres_body(y_ref, lm_ref, ls_ref, g_ref, b_ref, h_ref, o_ref):
    z = (y_ref[...] - lm_ref[...]) / ls_ref[...] * g_ref[...] + b_ref[...]
    o_ref[...] = jnp.maximum(z + h_ref[...], 0.0)


def _lnres(y, lm, ls, g, b, h):
    return pl.pallas_call(
        _lnres_body,
        out_shape=jax.ShapeDtypeStruct((N, H), jnp.float32),
    )(y, lm, ls, g, b, h)


_BE = 4000


def _edget_body(ea_ref, ew_ref, eb_ref, w0_ref, b0_ref, w1_ref, b1_ref,
                w2_ref, b2_ref, t0_ref, t1_ref, t2_ref):
    ea = jnp.dot(ea_ref[...], ew_ref[...], preferred_element_type=jnp.float32)
    ea = jnp.maximum(ea + eb_ref[...], 0.0)
    t0_ref[...] = jnp.dot(ea, w0_ref[...], preferred_element_type=jnp.float32) + b0_ref[...]
    t1_ref[...] = jnp.dot(ea, w1_ref[...], preferred_element_type=jnp.float32) + b1_ref[...]
    t2_ref[...] = jnp.dot(ea, w2_ref[...], preferred_element_type=jnp.float32) + b2_ref[...]


def _edge_t(edge_attr, ew, eb, ws, bs):
    grid = (E // _BE,)
    de = edge_attr.shape[1]
    wspec = pl.BlockSpec((de, H), lambda i: (0, 0))
    mspec = pl.BlockSpec((H, H), lambda i: (0, 0))
    bspec = pl.BlockSpec((1, H), lambda i: (0, 0))
    ospec = pl.BlockSpec((_BE, H), lambda i: (i, 0))
    return pl.pallas_call(
        _edget_body,
        grid=grid,
        in_specs=[pl.BlockSpec((_BE, de), lambda i: (i, 0)), wspec, bspec,
                  mspec, bspec, mspec, bspec, mspec, bspec],
        out_specs=[ospec, ospec, ospec],
        out_shape=[jax.ShapeDtypeStruct((E, H), jnp.float32)] * 3,
    )(edge_attr, ew, eb, ws[0], bs[0], ws[1], bs[1], ws[2], bs[2])


_NW = 32
_EPW = E // _NW
_KC = 80
_NCHUNK = _EPW // _KC
_RPT = 624
_RB = 208
_TAIL = N - 16 * _RPT


def _edge_sc_body(h_hbm, t_hbm, src_hbm, dst_hbm, out_hbm,
                  src_v, dst_v, t_v, rows_v, zb_v, sem, aggr_sh):
    c = lax.axis_index("c")
    s = lax.axis_index("s")

    def _zrow(i, carry):
        for c8 in range(8):
            zb_v[i, pl.ds(c8 * 16, 16)] = jnp.zeros((16,), jnp.float32)
        return carry

    lax.fori_loop(0, _RB, _zrow, 0)
    for j in range(_RPT // _RB):
        pltpu.sync_copy(zb_v, aggr_sh.at[pl.ds(s * _RPT + j * _RB, _RB)])

    @pl.when(s == 0)
    def _zero_tail():
        pltpu.sync_copy(zb_v.at[pl.ds(0, _TAIL)],
                        aggr_sh.at[pl.ds(16 * _RPT, _TAIL)])

    plsc.subcore_barrier()

    base0 = (c * 16 + s) * _EPW

    def _chunk(i, carry):
        base = base0 + i * _KC
        pltpu.sync_copy(src_hbm.at[pl.ds(base, _KC)], src_v)
        pltpu.sync_copy(dst_hbm.at[pl.ds(base, _KC)], dst_v)
        pltpu.sync_copy(t_hbm.at[pl.ds(base, _KC)], t_v)
        pltpu.async_copy(h_hbm.at[src_v], rows_v, sem).wait()

        def _row(r, rc):
            for c8 in range(8):
                sl = pl.ds(c8 * 16, 16)
                rows_v[r, sl] = jnp.maximum(rows_v[r, sl] + t_v[r, sl], 0.0)
            return rc

        lax.fori_loop(0, _KC, _row, 0)
        pltpu.sync_copy(rows_v, aggr_sh.at[dst_v], add=True)
        return carry

    lax.fori_loop(0, _NCHUNK, _chunk, 0)
    plsc.subcore_barrier()

    for j in range(_RPT // _RB):
        r0 = s * _RPT + j * _RB
        pltpu.sync_copy(aggr_sh.at[pl.ds(r0, _RB)], zb_v)
        pltpu.sync_copy(zb_v, out_hbm.at[c, pl.ds(r0, _RB)])

    @pl.when(s == 0)
    def _out_tail():
        pltpu.sync_copy(aggr_sh.at[pl.ds(16 * _RPT, _TAIL)],
                        zb_v.at[pl.ds(0, _TAIL)])
        pltpu.sync_copy(zb_v.at[pl.ds(0, _TAIL)],
                        out_hbm.at[c, pl.ds(16 * _RPT, _TAIL)])


def _edge_pass(h, t, src, dst):
    mesh = plsc.VectorSubcoreMesh(core_axis_name="c", subcore_axis_name="s",
                                  num_cores=2, num_subcores=16)
    f = pl.kernel(
        _edge_sc_body,
        out_type=jax.ShapeDtypeStruct((2, N, H), jnp.float32),
        mesh=mesh,
        scratch_types=[
            pltpu.VMEM((_KC,), jnp.int32),
            pltpu.VMEM((_KC,), jnp.int32),
            pltpu.VMEM((_KC, H), jnp.float32),
            pltpu.VMEM((_KC, H), jnp.float32),
            pltpu.VMEM((_RB, H), jnp.float32),
            pltpu.SemaphoreType.DMA,
            pltpu.VMEM_SHARED((N, H), jnp.float32),
        ],
    )
    return f(h, t, src, dst)


def _readout_body(h1_ref, h2_ref, h3_ref, bf_ref, gf_ref,
                  jw1_ref, jb1_ref, jw2t_ref, jb2_ref,
                  wa_ref, wb_ref, wc_ref, wd_ref, b1_ref,
                  o_ref, mx_ref):
    hs = (h1_ref[...], h2_ref[...], h3_ref[...])
    scores = []
    for h in hs:
        t = jnp.dot(h, jw1_ref[...], preferred_element_type=jnp.float32)
        t = jnp.maximum(t + jb1_ref[...], 0.0)
        sc = jnp.dot(t, jw2t_ref[...], preferred_element_type=jnp.float32)
        scores.append(sc + jb2_ref[...])
    msc = jnp.maximum(jnp.maximum(scores[0], scores[1]), scores[2])
    e0 = jnp.exp(scores[0] - msc)
    e1 = jnp.exp(scores[1] - msc)
    e2 = jnp.exp(scores[2] - msc)
    inv = 1.0 / (e0 + e1 + e2)
    xf = (e0 * hs[0] + e1 * hs[1] + e2 * hs[2]) * inv

    bf = bf_ref[...]
    gids = lax.broadcasted_iota(jnp.int32, (1, NG), 1)
    mask = jnp.where(bf == gids, 1.0, 0.0)
    dn = (((0,), (0,)), ((), ()))
    ssum = lax.dot_general(mask, xf, dn, preferred_element_type=jnp.float32,
                           precision=lax.Precision.HIGHEST)
    ones = jnp.ones_like(xf)
    cnt = lax.dot_general(mask, ones, dn, preferred_element_type=jnp.float32,
                          precision=lax.Precision.HIGHEST)

    def _mx(g, carry):
        w = jnp.where(bf == g, xf, -1e30)
        mx_ref[pl.ds(g, 1), :] = jnp.max(w, axis=0, keepdims=True)
        return carry

    lax.fori_loop(0, NG, _mx, 0)
    mx = jnp.where(cnt > 0.0, mx_ref[...], 0.0)
    mean = ssum / jnp.maximum(cnt, 1.0)

    pre = jnp.dot(mean, wa_ref[...], preferred_element_type=jnp.float32)
    pre = pre + jnp.dot(mx, wb_ref[...], preferred_element_type=jnp.float32)
    pre = pre + jnp.dot(ssum, wc_ref[...], preferred_element_type=jnp.float32)
    pre = pre + jnp.dot(gf_ref[...], wd_ref[...], preferred_element_type=jnp.float32)
    o_ref[...] = pre + b1_ref[...]


def _readout(h1, h2, h3, bf, gf, jw1, jb1, jw2t, jb2, wa, wb, wc, wd, b1):
    return pl.pallas_call(
        _readout_body,
        out_shape=jax.ShapeDtypeStruct((NG, H), jnp.float32),
        scratch_shapes=[pltpu.VMEM((NG, H), jnp.float32)],
    )(h1, h2, h3, bf, gf, jw1, jb1, jw2t, jb2, wa, wb, wc, wd, b1)


def _head_body(p_ref, m_ref, s_ref, g_ref, b_ref, w_ref, b2_ref, o_ref):
    z = jnp.maximum(
        (p_ref[...] - m_ref[...]) / s_ref[...] * g_ref[...] + b_ref[...], 0.0)
    o_ref[...] = jnp.dot(z, w_ref[...],
                         preferred_element_type=jnp.float32) + b2_ref[...]


def _head(pre, m, s, g, b, w2t, b2):
    return pl.pallas_call(
        _head_body,
        out_shape=jax.ShapeDtypeStruct((NG, H), jnp.float32),
    )(pre, m, s, g, b, w2t, b2)


def _row(v):
    return v.reshape(1, -1).astype(jnp.float32)


def kernel(x, edge_index, edge_attr, batch, graph_features, params):
    p = params
    src = edge_index[0]
    dst = edge_index[1]

    y0 = _mm_in(x, p['in_w'], _row(p['in_b']))
    s0 = jnp.sqrt(y0.var(axis=0) + 1e-5)
    h = _norm_relu(y0, _row(y0.mean(axis=0)), _row(s0),
                   _row(p['in_bn_g']), _row(p['in_bn_b']))

    ws = [bp['elin_w'] for bp in p['blocks']]
    bs = [_row(bp['elin_b']) for bp in p['blocks']]
    ts = _edge_t(edge_attr, p['e_w'], _row(p['e_b']), ws, bs)

    outs = []
    for l, bp in enumerate(p['blocks']):
        aggr = _edge_pass(h, ts[l], src, dst)
        scale = jnp.full((1, 1), 1.0, jnp.float32) + bp['eps']
        y1 = _zw1(h, aggr, scale, bp['w1'], _row(bp['b1']))
        s1 = jnp.sqrt(y1.var(axis=0) + 1e-5)
        y2 = _nw2(y1, _row(y1.mean(axis=0)), _row(s1),
                  _row(bp['bn_g']), _row(bp['bn_b']), bp['w2'], _row(bp['b2']))
        lm = y2.mean(axis=-1, keepdims=True)
        ls = jnp.sqrt(y2.var(axis=-1, keepdims=True) + 1e-5)
        h = _lnres(y2, lm, ls, _row(bp['ln_g']), _row(bp['ln_b']), h)
        outs.append(h)

    bf = batch.astype(jnp.int32).reshape(N, 1)
    gf = graph_features @ p['gf_w'] + p['gf_b']

    wa = p['cls_w1'][0:H]
    wb = p['cls_w1'][H:2 * H]
    wc = p['cls_w1'][2 * H:3 * H]
    wd = p['cls_w1'][3 * H:]

    jw2t = jnp.tile(p['jk_w2'], (1, H))
    w2t = jnp.tile(p['cls_w2'], (1, H))

    pre = _readout(outs[0], outs[1], outs[2], bf, gf,
                   p['jk_w1'], _row(p['jk_b1']), jw2t,
                   jnp.full((1, 1), 1.0, jnp.float32) * p['jk_b2'],
                   wa, wb, wc, wd, _row(p['cls_b1']))
    sb = jnp.sqrt(pre.var(axis=0) + 1e-5)
    out = _head(pre, _row(pre.mean(axis=0)), _row(sb),
                _row(p['cls_bn_g']), _row(p['cls_bn_b']), w2t,
                jnp.full((1, 1), 1.0, jnp.float32) * p['cls_b2'])
    return out[:, 0]
